# Initial kernel scaffold; baseline (speedup 1.0000x reference)
import jax
import jax.numpy as jnp
from jax.experimental import pallas as pl


def kernel(node_deg, edge_index, batch, conv0_w, conv0_b, eps0, bn0_g, bn0_b, conv1_w, conv1_b, eps1, bn1_g, bn1_b, conv2_w, conv2_b, eps2, bn2_g, bn2_b, cls_w1, cls_b1, cls_w2, cls_b2):
    raise NotImplementedError("write your pallas kernel here")


# speedup vs baseline: 2.8319x; 2.8319x over previous
import functools

import jax
import jax.numpy as jnp
from jax import lax
from jax.experimental import pallas as pl
from jax.experimental.pallas import tpu as pltpu
from jax.experimental.pallas import tpu_sc as plsc

N = 50000
D = 64
G = 128
NUM_CLASSES = 10
CLS_HIDDEN = 128

NPAD = 50176
HALF = 25088
ACC_ROWS = 26624
E = 800000
CHUNK = 128
NCHUNK = 392
EPT = NCHUNK * CHUNK
E_PAD = EPT * 16

BLK = 512
GRID = NPAD // BLK


def _leaky(x):
    return jnp.where(x >= 0, x, 0.01 * x)


def _sc_agg(h, src, dst):
    mesh = plsc.VectorSubcoreMesh(core_axis_name="c", subcore_axis_name="s")

    @functools.partial(
        pl.kernel,
        mesh=mesh,
        compiler_params=pltpu.CompilerParams(use_tc_tiling_on_sc=False),
        out_type=jax.ShapeDtypeStruct((NPAD, D), jnp.float32),
        scratch_types=[
            pltpu.VMEM((CHUNK,), jnp.int32),
            pltpu.VMEM((1, CHUNK), jnp.int32),
            pltpu.VMEM((CHUNK, D), jnp.float32),
            pltpu.VMEM_SHARED((ACC_ROWS, D), jnp.float32),
            pltpu.SemaphoreType.DMA,
        ],
    )
    def agg_kernel(h_hbm, src_hbm, dst_hbm, out_hbm, src_v, ldst_v, rows_v, acc, sem):
        c = lax.axis_index("c")
        s = lax.axis_index("s")
        lo = c * HALF
        hi = lo + HALF

        def zrow(r, carry):
            for k in range(D // 16):
                rows_v[r, pl.ds(16 * k, 16)] = jnp.zeros((16,), jnp.float32)
            return carry

        lax.fori_loop(0, CHUNK, zrow, 0)
        for j in range(ACC_ROWS // 16 // CHUNK):
            pltpu.sync_copy(rows_v, acc.at[pl.ds(s * (ACC_ROWS // 16) + j * CHUNK, CHUNK)])
        plsc.subcore_barrier()

        lane = lax.iota(jnp.int32, 16)

        def body(j, carry):
            off = s * EPT + j * CHUNK
            pltpu.sync_copy(src_hbm.at[pl.ds(off, CHUNK)], src_v)
            pltpu.sync_copy(dst_hbm.at[pl.ds(off, CHUNK)], ldst_v.at[0])
            for i in range(CHUNK // 16):
                d = ldst_v[0, pl.ds(16 * i, 16)]
                owned = (d >= lo) & (d < hi)
                trash = HALF + s * 96 + ((16 * i + lane) % 96)
                ldst_v[0, pl.ds(16 * i, 16)] = jnp.where(owned, d - lo, trash)
            pltpu.async_copy(h_hbm.at[src_v], rows_v, sem).wait()
            pltpu.sync_copy(rows_v, acc.at[ldst_v.at[0]], add=True)
            return carry

        lax.fori_loop(0, NCHUNK, body, 0)
        plsc.subcore_barrier()

        pltpu.sync_copy(
            acc.at[pl.ds(s * (HALF // 16), HALF // 16)],
            out_hbm.at[pl.ds(c * HALF + s * (HALF // 16), HALF // 16)],
        )

    return agg_kernel(h, src, dst)


def _xbuild(deg_f, bat_f):
    def kern(deg_ref, bat_ref, x_ref, px_ref):
        step = pl.program_id(0)
        d = deg_ref[...]
        cols = lax.broadcasted_iota(jnp.int32, (BLK, D), 1).astype(jnp.float32)
        x = (d == cols).astype(jnp.float32)
        x_ref[...] = x
        gcols = lax.broadcasted_iota(jnp.int32, (BLK, G), 1).astype(jnp.float32)
        p = (bat_ref[...] == gcols).astype(jnp.float32)

        @pl.when(step == 0)
        def _():
            px_ref[...] = jnp.zeros_like(px_ref)

        px_ref[...] += lax.dot_general(p, x, (((0,), (0,)), ((), ())),
                                       preferred_element_type=jnp.float32, precision=lax.Precision.HIGHEST)

    return pl.pallas_call(
        kern,
        grid=(GRID,),
        in_specs=[pl.BlockSpec((BLK, 1), lambda i: (i, 0)),
                  pl.BlockSpec((BLK, 1), lambda i: (i, 0))],
        out_specs=[pl.BlockSpec((BLK, D), lambda i: (i, 0)),
                   pl.BlockSpec((G, D), lambda i: (0, 0))],
        out_shape=[jax.ShapeDtypeStruct((NPAD, D), jnp.float32),
                   jax.ShapeDtypeStruct((G, D), jnp.float32)],
    )(deg_f, bat_f)


def _stats(h, agg, eps11):
    def kern(h_ref, a_ref, e_ref, su_ref, c_ref):
        step = pl.program_id(0)
        u = (1.0 + e_ref[0, 0]) * h_ref[...] + a_ref[...]
        rows = lax.broadcasted_iota(jnp.int32, (BLK, 1), 0) + step * BLK
        um = jnp.where(rows < N, u, 0.0)

        @pl.when(step == 0)
        def _():
            su_ref[...] = jnp.zeros_like(su_ref)
            c_ref[...] = jnp.zeros_like(c_ref)

        row0 = (lax.broadcasted_iota(jnp.int32, (8, D), 0) == 0).astype(jnp.float32)
        su_ref[...] += row0 * jnp.sum(um, axis=0, keepdims=True)
        c_ref[...] += lax.dot_general(um, um, (((0,), (0,)), ((), ())),
                                      preferred_element_type=jnp.float32, precision=lax.Precision.HIGHEST)

    return pl.pallas_call(
        kern,
        grid=(GRID,),
        in_specs=[pl.BlockSpec((BLK, D), lambda i: (i, 0)),
                  pl.BlockSpec((BLK, D), lambda i: (i, 0)),
                  pl.BlockSpec((1, 1), lambda i: (0, 0))],
        out_specs=[pl.BlockSpec((8, D), lambda i: (0, 0)),
                   pl.BlockSpec((D, D), lambda i: (0, 0))],
        out_shape=[jax.ShapeDtypeStruct((8, D), jnp.float32),
                   jax.ShapeDtypeStruct((D, D), jnp.float32)],
    )(h, agg, eps11)


def _layer_out(h, agg, bat_f, eps11, su, cmat, w, g, bb):
    def kern(h_ref, a_ref, bat_ref, e_ref, su_ref, c_ref, w_ref, g_ref, bb_ref,
             hn_ref, p_ref):
        step = pl.program_id(0)
        w_ = w_ref[...]
        svec = su_ref[...][0:1, :]
        mean = jnp.dot(svec, w_, preferred_element_type=jnp.float32, precision=lax.Precision.HIGHEST) / N
        m = jnp.dot(c_ref[...], w_, preferred_element_type=jnp.float32, precision=lax.Precision.HIGHEST)
        ey2 = jnp.sum(w_ * m, axis=0, keepdims=True) / N
        var = ey2 - mean * mean
        scale = g_ref[...] * lax.rsqrt(var + 1e-5)
        w2 = w_ * scale
        b2 = bb_ref[...] - mean * scale
        u = (1.0 + e_ref[0, 0]) * h_ref[...] + a_ref[...]
        z = _leaky(jnp.dot(u, w2, preferred_element_type=jnp.float32, precision=lax.Precision.HIGHEST) + b2)
        hn_ref[...] = z
        gcols = lax.broadcasted_iota(jnp.int32, (BLK, G), 1).astype(jnp.float32)
        p = (bat_ref[...] == gcols).astype(jnp.float32)

        @pl.when(step == 0)
        def _():
            p_ref[...] = jnp.zeros_like(p_ref)

        p_ref[...] += lax.dot_general(p, z, (((0,), (0,)), ((), ())),
                                      preferred_element_type=jnp.float32, precision=lax.Precision.HIGHEST)

    return pl.pallas_call(
        kern,
        grid=(GRID,),
        in_specs=[pl.BlockSpec((BLK, D), lambda i: (i, 0)),
                  pl.BlockSpec((BLK, D), lambda i: (i, 0)),
                  pl.BlockSpec((BLK, 1), lambda i: (i, 0)),
                  pl.BlockSpec((1, 1), lambda i: (0, 0)),
                  pl.BlockSpec((8, D), lambda i: (0, 0)),
                  pl.BlockSpec((D, D), lambda i: (0, 0)),
                  pl.BlockSpec((D, D), lambda i: (0, 0)),
                  pl.BlockSpec((1, D), lambda i: (0, 0)),
                  pl.BlockSpec((1, D), lambda i: (0, 0))],
        out_specs=[pl.BlockSpec((BLK, D), lambda i: (i, 0)),
                   pl.BlockSpec((G, D), lambda i: (0, 0))],
        out_shape=[jax.ShapeDtypeStruct((NPAD, D), jnp.float32),
                   jax.ShapeDtypeStruct((G, D), jnp.float32)],
    )(h, agg, bat_f, eps11, su, cmat, w, g, bb)


def _head(px, p1, p2, p3, w1, b1, w2, b2):
    def kern(px_ref, p1_ref, p2_ref, p3_ref, w1_ref, b1_ref, w2_ref, b2_ref, o_ref):
        o = jnp.dot(_leaky(px_ref[...]), w1_ref[0:D, :],
                    preferred_element_type=jnp.float32, precision=lax.Precision.HIGHEST)
        o += jnp.dot(_leaky(p1_ref[...]), w1_ref[D:2 * D, :],
                     preferred_element_type=jnp.float32, precision=lax.Precision.HIGHEST)
        o += jnp.dot(_leaky(p2_ref[...]), w1_ref[2 * D:3 * D, :],
                     preferred_element_type=jnp.float32, precision=lax.Precision.HIGHEST)
        o += jnp.dot(_leaky(p3_ref[...]), w1_ref[3 * D:4 * D, :],
                     preferred_element_type=jnp.float32, precision=lax.Precision.HIGHEST)
        o = _leaky(o + b1_ref[...])
        o_ref[...] = jnp.dot(o, w2_ref[...], preferred_element_type=jnp.float32, precision=lax.Precision.HIGHEST) \
            + b2_ref[...]

    return pl.pallas_call(
        kern,
        out_shape=jax.ShapeDtypeStruct((G, NUM_CLASSES), jnp.float32),
    )(px, p1, p2, p3, w1, b1, w2, b2)


def kernel(node_deg, edge_index, batch,
           conv0_w, conv0_b, eps0, bn0_g, bn0_b,
           conv1_w, conv1_b, eps1, bn1_g, bn1_b,
           conv2_w, conv2_b, eps2, bn2_g, bn2_b,
           cls_w1, cls_b1, cls_w2, cls_b2):
    f32 = jnp.float32
    deg = node_deg.astype(jnp.int32)
    deg_f = jnp.concatenate(
        [deg, jnp.full((NPAD - N,), -1, jnp.int32)]).astype(f32).reshape(NPAD, 1)
    bat_f = jnp.concatenate(
        [batch.astype(jnp.int32), jnp.full((NPAD - N,), G, jnp.int32)]
    ).astype(f32).reshape(NPAD, 1)
    pad_n = E_PAD - E
    pad_src = (jnp.arange(pad_n, dtype=jnp.int32) * 17) % N
    pad_dst = jnp.full((pad_n,), jnp.int32(1 << 28))
    src = jnp.concatenate([edge_index[0].astype(jnp.int32), pad_src])
    dst = jnp.concatenate([edge_index[1].astype(jnp.int32), pad_dst])

    x, px = _xbuild(deg_f, bat_f)
    params = [(conv0_w, eps0, bn0_g, bn0_b),
              (conv1_w, eps1, bn1_g, bn1_b),
              (conv2_w, eps2, bn2_g, bn2_b)]
    h = x
    pooled = [px]
    for (w, eps, g, bb) in params:
        e11 = jnp.reshape(eps.astype(f32), (1, 1))
        aggv = _sc_agg(h, src, dst)
        su, cmat = _stats(h, aggv, e11)
        h, pk = _layer_out(h, aggv, bat_f, e11, su, cmat, w,
                           g.reshape(1, D), bb.reshape(1, D))
        pooled.append(pk)
    return _head(pooled[0], pooled[1], pooled[2], pooled[3],
                 cls_w1, cls_b1.reshape(1, CLS_HIDDEN),
                 cls_w2, cls_b2.reshape(1, NUM_CLASSES))


# speedup vs baseline: 5.7265x; 2.0222x over previous
import functools

import jax
import jax.numpy as jnp
from jax import lax
from jax.experimental import pallas as pl
from jax.experimental.pallas import tpu as pltpu
from jax.experimental.pallas import tpu_sc as plsc

N = 50000
D = 64
G = 128
NUM_CLASSES = 10
CLS_HIDDEN = 128

NPAD = 50176
HALF = 25088
ACC_ROWS = 26624
E = 800000
CHUNK = 128
SUBCH = 8
NBLK = 49
EPT = NBLK * SUBCH * CHUNK
E_PAD = EPT * 16
E2D = E_PAD // CHUNK
TROWS = EPT // CHUNK

BLK = 512
GRID = NPAD // BLK


def _leaky(x):
    return jnp.where(x >= 0, x, 0.01 * x)


def _sc_agg(h, src2, dst2):
    mesh = plsc.VectorSubcoreMesh(core_axis_name="c", subcore_axis_name="s")

    @functools.partial(
        pl.kernel,
        mesh=mesh,
        compiler_params=pltpu.CompilerParams(use_tc_tiling_on_sc=False),
        out_type=jax.ShapeDtypeStruct((NPAD, D), jnp.float32),
        scratch_types=[
            pltpu.VMEM((2, SUBCH, CHUNK), jnp.int32),
            pltpu.VMEM((2, SUBCH, CHUNK), jnp.int32),
            pltpu.VMEM((2, CHUNK, D), jnp.float32),
            pltpu.VMEM_SHARED((ACC_ROWS, D), jnp.float32),
            pltpu.SemaphoreType.DMA,
            pltpu.SemaphoreType.DMA,
            pltpu.SemaphoreType.DMA,
            pltpu.SemaphoreType.DMA,
            pltpu.SemaphoreType.DMA,
        ],
    )
    def agg_kernel(h_hbm, src_hbm, dst_hbm, out_hbm, eidx, ldst, rows4, acc,
                   esem, g0, g1, s0, s1):
        gsems = [g0, g1]
        ssems = [s0, s1]
        c = lax.axis_index("c")
        s = lax.axis_index("s")
        lo = c * HALF
        hi = lo + HALF
        lane = lax.iota(jnp.int32, 16)

        def gather_start(eb, i, b):
            pltpu.async_copy(h_hbm.at[eidx.at[eb, i]], rows4.at[b], gsems[b])

        def gather_wait(b):
            pltpu.make_async_copy(h_hbm.at[eidx.at[0, 0]], rows4.at[b],
                                  gsems[b]).wait()

        def scatter_start(eb, i, b):
            pltpu.async_copy(rows4.at[b], acc.at[ldst.at[eb, i]], ssems[b],
                             add=True)

        def scatter_wait(b):
            pltpu.make_async_copy(rows4.at[b], acc.at[ldst.at[0, 0]],
                                  ssems[b]).wait()

        def edge_start(eb, row0):
            pltpu.async_copy(src_hbm.at[pl.ds(row0, SUBCH)], eidx.at[eb], esem)
            pltpu.async_copy(dst_hbm.at[pl.ds(row0, SUBCH)], ldst.at[eb], esem)

        def edge_wait():
            pltpu.make_async_copy(src_hbm.at[pl.ds(0, SUBCH)], eidx.at[0],
                                  esem).wait()
            pltpu.make_async_copy(dst_hbm.at[pl.ds(0, SUBCH)], ldst.at[0],
                                  esem).wait()

        def zrow(r, carry):
            for k in range(D // 16):
                rows4[0, r, pl.ds(16 * k, 16)] = jnp.zeros((16,), jnp.float32)
            return carry

        lax.fori_loop(0, CHUNK, zrow, 0)
        for j in range(ACC_ROWS // 16 // CHUNK):
            pltpu.sync_copy(rows4.at[0],
                            acc.at[pl.ds(s * (ACC_ROWS // 16) + j * CHUNK, CHUNK)])
        plsc.subcore_barrier()

        edge_start(0, s * TROWS)

        def outer(blk, carry):
            eb = lax.rem(blk, 2)
            edge_wait()
            for i in range(SUBCH):
                for k in range(CHUNK // 16):
                    dv = ldst[eb, i, pl.ds(16 * k, 16)]
                    owned = (dv >= lo) & (dv < hi)
                    tr = HALF + s * 96 + ((i * CHUNK + 16 * k + lane) % 96)
                    ldst[eb, i, pl.ds(16 * k, 16)] = jnp.where(owned, dv - lo, tr)
            for i in range(SUBCH):
                b = i % 2
                pb = 1 - b
                if i >= 2:
                    scatter_wait(b)
                else:
                    @pl.when(blk > 0)
                    def _():
                        scatter_wait(b)
                gather_start(eb, i, b)
                if i >= 1:
                    gather_wait(pb)
                    scatter_start(eb, i - 1, pb)
                else:
                    @pl.when(blk > 0)
                    def _():
                        gather_wait(pb)
                        scatter_start(1 - eb, SUBCH - 1, pb)
                if i == 2:
                    @pl.when(blk + 1 < NBLK)
                    def _():
                        edge_start(1 - eb, s * TROWS + (blk + 1) * SUBCH)
            return carry

        lax.fori_loop(0, NBLK, outer, 0)

        gather_wait(1)
        scatter_start(0, SUBCH - 1, 1)
        scatter_wait(0)
        scatter_wait(1)
        plsc.subcore_barrier()

        pltpu.sync_copy(
            acc.at[pl.ds(s * (HALF // 16), HALF // 16)],
            out_hbm.at[pl.ds(c * HALF + s * (HALF // 16), HALF // 16)],
        )

    return agg_kernel(h, src2, dst2)


def _xbuild(deg_f, bat_f):
    def kern(deg_ref, bat_ref, x_ref, px_ref):
        step = pl.program_id(0)
        d = deg_ref[...]
        cols = lax.broadcasted_iota(jnp.int32, (BLK, D), 1).astype(jnp.float32)
        x = (d == cols).astype(jnp.float32)
        x_ref[...] = x
        gcols = lax.broadcasted_iota(jnp.int32, (BLK, G), 1).astype(jnp.float32)
        p = (bat_ref[...] == gcols).astype(jnp.float32)

        @pl.when(step == 0)
        def _():
            px_ref[...] = jnp.zeros_like(px_ref)

        px_ref[...] += lax.dot_general(p, x, (((0,), (0,)), ((), ())),
                                       preferred_element_type=jnp.float32, precision=lax.Precision.HIGHEST)

    return pl.pallas_call(
        kern,
        grid=(GRID,),
        in_specs=[pl.BlockSpec((BLK, 1), lambda i: (i, 0)),
                  pl.BlockSpec((BLK, 1), lambda i: (i, 0))],
        out_specs=[pl.BlockSpec((BLK, D), lambda i: (i, 0)),
                   pl.BlockSpec((G, D), lambda i: (0, 0))],
        out_shape=[jax.ShapeDtypeStruct((NPAD, D), jnp.float32),
                   jax.ShapeDtypeStruct((G, D), jnp.float32)],
    )(deg_f, bat_f)


def _stats(h, agg, eps11):
    def kern(h_ref, a_ref, e_ref, su_ref, c_ref):
        step = pl.program_id(0)
        u = (1.0 + e_ref[0, 0]) * h_ref[...] + a_ref[...]
        rows = lax.broadcasted_iota(jnp.int32, (BLK, 1), 0) + step * BLK
        um = jnp.where(rows < N, u, 0.0)

        @pl.when(step == 0)
        def _():
            su_ref[...] = jnp.zeros_like(su_ref)
            c_ref[...] = jnp.zeros_like(c_ref)

        row0 = (lax.broadcasted_iota(jnp.int32, (8, D), 0) == 0).astype(jnp.float32)
        su_ref[...] += row0 * jnp.sum(um, axis=0, keepdims=True)
        c_ref[...] += lax.dot_general(um, um, (((0,), (0,)), ((), ())),
                                      preferred_element_type=jnp.float32, precision=lax.Precision.HIGHEST)

    return pl.pallas_call(
        kern,
        grid=(GRID,),
        in_specs=[pl.BlockSpec((BLK, D), lambda i: (i, 0)),
                  pl.BlockSpec((BLK, D), lambda i: (i, 0)),
                  pl.BlockSpec((1, 1), lambda i: (0, 0))],
        out_specs=[pl.BlockSpec((8, D), lambda i: (0, 0)),
                   pl.BlockSpec((D, D), lambda i: (0, 0))],
        out_shape=[jax.ShapeDtypeStruct((8, D), jnp.float32),
                   jax.ShapeDtypeStruct((D, D), jnp.float32)],
    )(h, agg, eps11)


def _layer_out(h, agg, bat_f, eps11, su, cmat, w, g, bb):
    def kern(h_ref, a_ref, bat_ref, e_ref, su_ref, c_ref, w_ref, g_ref, bb_ref,
             hn_ref, p_ref):
        step = pl.program_id(0)
        w_ = w_ref[...]
        svec = su_ref[...][0:1, :]
        mean = jnp.dot(svec, w_, preferred_element_type=jnp.float32, precision=lax.Precision.HIGHEST) / N
        m = jnp.dot(c_ref[...], w_, preferred_element_type=jnp.float32, precision=lax.Precision.HIGHEST)
        ey2 = jnp.sum(w_ * m, axis=0, keepdims=True) / N
        var = ey2 - mean * mean
        scale = g_ref[...] * lax.rsqrt(var + 1e-5)
        w2 = w_ * scale
        b2 = bb_ref[...] - mean * scale
        u = (1.0 + e_ref[0, 0]) * h_ref[...] + a_ref[...]
        z = _leaky(jnp.dot(u, w2, preferred_element_type=jnp.float32, precision=lax.Precision.HIGHEST) + b2)
        hn_ref[...] = z
        gcols = lax.broadcasted_iota(jnp.int32, (BLK, G), 1).astype(jnp.float32)
        p = (bat_ref[...] == gcols).astype(jnp.float32)

        @pl.when(step == 0)
        def _():
            p_ref[...] = jnp.zeros_like(p_ref)

        p_ref[...] += lax.dot_general(p, z, (((0,), (0,)), ((), ())),
                                      preferred_element_type=jnp.float32, precision=lax.Precision.HIGHEST)

    return pl.pallas_call(
        kern,
        grid=(GRID,),
        in_specs=[pl.BlockSpec((BLK, D), lambda i: (i, 0)),
                  pl.BlockSpec((BLK, D), lambda i: (i, 0)),
                  pl.BlockSpec((BLK, 1), lambda i: (i, 0)),
                  pl.BlockSpec((1, 1), lambda i: (0, 0)),
                  pl.BlockSpec((8, D), lambda i: (0, 0)),
                  pl.BlockSpec((D, D), lambda i: (0, 0)),
                  pl.BlockSpec((D, D), lambda i: (0, 0)),
                  pl.BlockSpec((1, D), lambda i: (0, 0)),
                  pl.BlockSpec((1, D), lambda i: (0, 0))],
        out_specs=[pl.BlockSpec((BLK, D), lambda i: (i, 0)),
                   pl.BlockSpec((G, D), lambda i: (0, 0))],
        out_shape=[jax.ShapeDtypeStruct((NPAD, D), jnp.float32),
                   jax.ShapeDtypeStruct((G, D), jnp.float32)],
    )(h, agg, bat_f, eps11, su, cmat, w, g, bb)


def _head(px, p1, p2, p3, w1, b1, w2, b2):
    def kern(px_ref, p1_ref, p2_ref, p3_ref, w1_ref, b1_ref, w2_ref, b2_ref, o_ref):
        o = jnp.dot(_leaky(px_ref[...]), w1_ref[0:D, :],
                    preferred_element_type=jnp.float32, precision=lax.Precision.HIGHEST)
        o += jnp.dot(_leaky(p1_ref[...]), w1_ref[D:2 * D, :],
                     preferred_element_type=jnp.float32, precision=lax.Precision.HIGHEST)
        o += jnp.dot(_leaky(p2_ref[...]), w1_ref[2 * D:3 * D, :],
                     preferred_element_type=jnp.float32, precision=lax.Precision.HIGHEST)
        o += jnp.dot(_leaky(p3_ref[...]), w1_ref[3 * D:4 * D, :],
                     preferred_element_type=jnp.float32, precision=lax.Precision.HIGHEST)
        o = _leaky(o + b1_ref[...])
        o_ref[...] = jnp.dot(o, w2_ref[...], preferred_element_type=jnp.float32, precision=lax.Precision.HIGHEST) \
            + b2_ref[...]

    return pl.pallas_call(
        kern,
        out_shape=jax.ShapeDtypeStruct((G, NUM_CLASSES), jnp.float32),
    )(px, p1, p2, p3, w1, b1, w2, b2)


def kernel(node_deg, edge_index, batch,
           conv0_w, conv0_b, eps0, bn0_g, bn0_b,
           conv1_w, conv1_b, eps1, bn1_g, bn1_b,
           conv2_w, conv2_b, eps2, bn2_g, bn2_b,
           cls_w1, cls_b1, cls_w2, cls_b2):
    f32 = jnp.float32
    deg = node_deg.astype(jnp.int32)
    deg_f = jnp.concatenate(
        [deg, jnp.full((NPAD - N,), -1, jnp.int32)]).astype(f32).reshape(NPAD, 1)
    bat_f = jnp.concatenate(
        [batch.astype(jnp.int32), jnp.full((NPAD - N,), G, jnp.int32)]
    ).astype(f32).reshape(NPAD, 1)
    pad_n = E_PAD - E
    pad_src = (jnp.arange(pad_n, dtype=jnp.int32) * 17) % N
    pad_dst = jnp.full((pad_n,), jnp.int32(1 << 28))
    src = jnp.concatenate([edge_index[0].astype(jnp.int32), pad_src]).reshape(E2D, CHUNK)
    dst = jnp.concatenate([edge_index[1].astype(jnp.int32), pad_dst]).reshape(E2D, CHUNK)

    x, px = _xbuild(deg_f, bat_f)
    params = [(conv0_w, eps0, bn0_g, bn0_b),
              (conv1_w, eps1, bn1_g, bn1_b),
              (conv2_w, eps2, bn2_g, bn2_b)]
    h = x
    pooled = [px]
    for (w, eps, g, bb) in params:
        e11 = jnp.reshape(eps.astype(f32), (1, 1))
        aggv = _sc_agg(h, src, dst)
        su, cmat = _stats(h, aggv, e11)
        h, pk = _layer_out(h, aggv, bat_f, e11, su, cmat, w,
                           g.reshape(1, D), bb.reshape(1, D))
        pooled.append(pk)
    return _head(pooled[0], pooled[1], pooled[2], pooled[3],
                 cls_w1, cls_b1.reshape(1, CLS_HIDDEN),
                 cls_w2, cls_b2.reshape(1, NUM_CLASSES))


# speedup vs baseline: 6.1875x; 1.0805x over previous
import functools

import jax
import jax.numpy as jnp
from jax import lax
from jax.experimental import pallas as pl
from jax.experimental.pallas import tpu as pltpu
from jax.experimental.pallas import tpu_sc as plsc

N = 50000
D = 64
G = 128
NUM_CLASSES = 10
CLS_HIDDEN = 128

NPAD = 50176
HALF = 25088
ACC_ROWS = 26624
E = 800000
CHUNK = 128
SUBCH = 8
NBLK = 49
EPT = NBLK * SUBCH * CHUNK
E_PAD = EPT * 16
E2D = E_PAD // CHUNK
TROWS = EPT // CHUNK

BLK = 512
GRID = NPAD // BLK


def _leaky(x):
    return jnp.where(x >= 0, x, 0.01 * x)


def _sc_agg(h, src2, dst2):
    mesh = plsc.VectorSubcoreMesh(core_axis_name="c", subcore_axis_name="s")

    @functools.partial(
        pl.kernel,
        mesh=mesh,
        compiler_params=pltpu.CompilerParams(use_tc_tiling_on_sc=False),
        out_type=jax.ShapeDtypeStruct((NPAD, D), jnp.float32),
        scratch_types=[
            pltpu.VMEM((2, SUBCH, CHUNK), jnp.int32),
            pltpu.VMEM((2, SUBCH, CHUNK), jnp.int32),
            pltpu.VMEM((2, CHUNK, D), jnp.float32),
            pltpu.VMEM_SHARED((ACC_ROWS, D), jnp.float32),
            pltpu.SemaphoreType.DMA,
            pltpu.SemaphoreType.DMA,
            pltpu.SemaphoreType.DMA,
            pltpu.SemaphoreType.DMA,
            pltpu.SemaphoreType.DMA,
        ],
    )
    def agg_kernel(h_hbm, src_hbm, dst_hbm, out_hbm, eidx, ldst, rows4, acc,
                   esem, g0, g1, s0, s1):
        gsems = [g0, g1]
        ssems = [s0, s1]
        c = lax.axis_index("c")
        s = lax.axis_index("s")
        lo = c * HALF
        hi = lo + HALF
        lane = lax.iota(jnp.int32, 16)

        def gather_start(eb, i, b):
            pltpu.async_copy(h_hbm.at[eidx.at[eb, i]], rows4.at[b], gsems[b])

        def gather_wait(b):
            pltpu.make_async_copy(h_hbm.at[eidx.at[0, 0]], rows4.at[b],
                                  gsems[b]).wait()

        def scatter_start(eb, i, b):
            pltpu.async_copy(rows4.at[b], acc.at[ldst.at[eb, i]], ssems[b],
                             add=True)

        def scatter_wait(b):
            pltpu.make_async_copy(rows4.at[b], acc.at[ldst.at[0, 0]],
                                  ssems[b]).wait()

        def edge_start(eb, row0):
            pltpu.async_copy(src_hbm.at[pl.ds(row0, SUBCH)], eidx.at[eb], esem)
            pltpu.async_copy(dst_hbm.at[pl.ds(row0, SUBCH)], ldst.at[eb], esem)

        def edge_wait():
            pltpu.make_async_copy(src_hbm.at[pl.ds(0, SUBCH)], eidx.at[0],
                                  esem).wait()
            pltpu.make_async_copy(dst_hbm.at[pl.ds(0, SUBCH)], ldst.at[0],
                                  esem).wait()

        def zrow(r, carry):
            for k in range(D // 16):
                rows4[0, r, pl.ds(16 * k, 16)] = jnp.zeros((16,), jnp.float32)
            return carry

        lax.fori_loop(0, CHUNK, zrow, 0)
        for j in range(ACC_ROWS // 16 // CHUNK):
            pltpu.sync_copy(rows4.at[0],
                            acc.at[pl.ds(s * (ACC_ROWS // 16) + j * CHUNK, CHUNK)])
        plsc.subcore_barrier()

        edge_start(0, s * TROWS)

        def outer(blk, carry):
            eb = lax.rem(blk, 2)
            edge_wait()
            for i in range(SUBCH):
                for k in range(CHUNK // 16):
                    dv = ldst[eb, i, pl.ds(16 * k, 16)]
                    owned = (dv >= lo) & (dv < hi)
                    tr = HALF + s * 96 + ((i * CHUNK + 16 * k + lane) % 96)
                    ldst[eb, i, pl.ds(16 * k, 16)] = jnp.where(owned, dv - lo, tr)
            for i in range(SUBCH):
                b = i % 2
                pb = 1 - b
                if i >= 2:
                    scatter_wait(b)
                else:
                    @pl.when(blk > 0)
                    def _():
                        scatter_wait(b)
                gather_start(eb, i, b)
                if i >= 1:
                    gather_wait(pb)
                    scatter_start(eb, i - 1, pb)
                else:
                    @pl.when(blk > 0)
                    def _():
                        gather_wait(pb)
                        scatter_start(1 - eb, SUBCH - 1, pb)
                if i == 2:
                    @pl.when(blk + 1 < NBLK)
                    def _():
                        edge_start(1 - eb, s * TROWS + (blk + 1) * SUBCH)
            return carry

        lax.fori_loop(0, NBLK, outer, 0)

        gather_wait(1)
        scatter_start(0, SUBCH - 1, 1)
        scatter_wait(0)
        scatter_wait(1)
        plsc.subcore_barrier()

        pltpu.sync_copy(
            acc.at[pl.ds(s * (HALF // 16), HALF // 16)],
            out_hbm.at[pl.ds(c * HALF + s * (HALF // 16), HALF // 16)],
        )

    return agg_kernel(h, src2, dst2)


def _xbuild(deg_f, bat_f):
    def kern(deg_ref, bat_ref, x_ref, px_ref):
        step = pl.program_id(0)
        d = deg_ref[...]
        cols = lax.broadcasted_iota(jnp.int32, (BLK, D), 1).astype(jnp.float32)
        x = (d == cols).astype(jnp.float32)
        x_ref[...] = x
        gcols = lax.broadcasted_iota(jnp.int32, (BLK, G), 1).astype(jnp.float32)
        p = (bat_ref[...] == gcols).astype(jnp.float32)

        @pl.when(step == 0)
        def _():
            px_ref[...] = jnp.zeros_like(px_ref)

        px_ref[...] += lax.dot_general(p, x, (((0,), (0,)), ((), ())),
                                       preferred_element_type=jnp.float32, precision=lax.Precision.HIGHEST)

    return pl.pallas_call(
        kern,
        grid=(GRID,),
        in_specs=[pl.BlockSpec((BLK, 1), lambda i: (i, 0)),
                  pl.BlockSpec((BLK, 1), lambda i: (i, 0))],
        out_specs=[pl.BlockSpec((BLK, D), lambda i: (i, 0)),
                   pl.BlockSpec((G, D), lambda i: (0, 0))],
        out_shape=[jax.ShapeDtypeStruct((NPAD, D), jnp.float32),
                   jax.ShapeDtypeStruct((G, D), jnp.float32)],
    )(deg_f, bat_f)


def _stats(h, agg, eps11, w):
    def kern(h_ref, a_ref, e_ref, w_ref, y_ref, mom_ref):
        step = pl.program_id(0)
        u = (1.0 + e_ref[0, 0]) * h_ref[...] + a_ref[...]
        y = jnp.dot(u, w_ref[...], preferred_element_type=jnp.float32,
                    precision=lax.Precision.HIGHEST)
        y_ref[...] = y
        rows = lax.broadcasted_iota(jnp.int32, (BLK, 1), 0) + step * BLK
        ym = jnp.where(rows < N, y, 0.0)

        @pl.when(step == 0)
        def _():
            mom_ref[...] = jnp.zeros_like(mom_ref)

        riota = lax.broadcasted_iota(jnp.int32, (8, D), 0)
        row0 = (riota == 0).astype(jnp.float32)
        row1 = (riota == 1).astype(jnp.float32)
        mom_ref[...] += (row0 * jnp.sum(ym, axis=0, keepdims=True)
                         + row1 * jnp.sum(ym * ym, axis=0, keepdims=True))

    return pl.pallas_call(
        kern,
        grid=(GRID,),
        in_specs=[pl.BlockSpec((BLK, D), lambda i: (i, 0)),
                  pl.BlockSpec((BLK, D), lambda i: (i, 0)),
                  pl.BlockSpec((1, 1), lambda i: (0, 0)),
                  pl.BlockSpec((D, D), lambda i: (0, 0))],
        out_specs=[pl.BlockSpec((BLK, D), lambda i: (i, 0)),
                   pl.BlockSpec((8, D), lambda i: (0, 0))],
        out_shape=[jax.ShapeDtypeStruct((NPAD, D), jnp.float32),
                   jax.ShapeDtypeStruct((8, D), jnp.float32)],
    )(h, agg, eps11, w)


def _layer_out(y, bat_f, mom, g, bb):
    def kern(y_ref, bat_ref, mom_ref, g_ref, bb_ref, hn_ref, p_ref):
        step = pl.program_id(0)
        m = mom_ref[...]
        mean = m[0:1, :] / N
        var = m[1:2, :] / N - mean * mean
        scale = g_ref[...] * lax.rsqrt(var + 1e-5)
        z = _leaky((y_ref[...] - mean) * scale + bb_ref[...])
        hn_ref[...] = z
        gcols = lax.broadcasted_iota(jnp.int32, (BLK, G), 1).astype(jnp.float32)
        p = (bat_ref[...] == gcols).astype(jnp.float32)

        @pl.when(step == 0)
        def _():
            p_ref[...] = jnp.zeros_like(p_ref)

        p_ref[...] += lax.dot_general(p, z, (((0,), (0,)), ((), ())),
                                      preferred_element_type=jnp.float32,
                                      precision=lax.Precision.HIGHEST)

    return pl.pallas_call(
        kern,
        grid=(GRID,),
        in_specs=[pl.BlockSpec((BLK, D), lambda i: (i, 0)),
                  pl.BlockSpec((BLK, 1), lambda i: (i, 0)),
                  pl.BlockSpec((8, D), lambda i: (0, 0)),
                  pl.BlockSpec((1, D), lambda i: (0, 0)),
                  pl.BlockSpec((1, D), lambda i: (0, 0))],
        out_specs=[pl.BlockSpec((BLK, D), lambda i: (i, 0)),
                   pl.BlockSpec((G, D), lambda i: (0, 0))],
        out_shape=[jax.ShapeDtypeStruct((NPAD, D), jnp.float32),
                   jax.ShapeDtypeStruct((G, D), jnp.float32)],
    )(y, bat_f, mom, g, bb)


def _head(px, p1, p2, p3, w1, b1, w2, b2):
    def kern(px_ref, p1_ref, p2_ref, p3_ref, w1_ref, b1_ref, w2_ref, b2_ref, o_ref):
        o = jnp.dot(_leaky(px_ref[...]), w1_ref[0:D, :],
                    preferred_element_type=jnp.float32, precision=lax.Precision.HIGHEST)
        o += jnp.dot(_leaky(p1_ref[...]), w1_ref[D:2 * D, :],
                     preferred_element_type=jnp.float32, precision=lax.Precision.HIGHEST)
        o += jnp.dot(_leaky(p2_ref[...]), w1_ref[2 * D:3 * D, :],
                     preferred_element_type=jnp.float32, precision=lax.Precision.HIGHEST)
        o += jnp.dot(_leaky(p3_ref[...]), w1_ref[3 * D:4 * D, :],
                     preferred_element_type=jnp.float32, precision=lax.Precision.HIGHEST)
        o = _leaky(o + b1_ref[...])
        o_ref[...] = jnp.dot(o, w2_ref[...], preferred_element_type=jnp.float32, precision=lax.Precision.HIGHEST) \
            + b2_ref[...]

    return pl.pallas_call(
        kern,
        out_shape=jax.ShapeDtypeStruct((G, NUM_CLASSES), jnp.float32),
    )(px, p1, p2, p3, w1, b1, w2, b2)


def kernel(node_deg, edge_index, batch,
           conv0_w, conv0_b, eps0, bn0_g, bn0_b,
           conv1_w, conv1_b, eps1, bn1_g, bn1_b,
           conv2_w, conv2_b, eps2, bn2_g, bn2_b,
           cls_w1, cls_b1, cls_w2, cls_b2):
    f32 = jnp.float32
    deg = node_deg.astype(jnp.int32)
    deg_f = jnp.concatenate(
        [deg, jnp.full((NPAD - N,), -1, jnp.int32)]).astype(f32).reshape(NPAD, 1)
    bat_f = jnp.concatenate(
        [batch.astype(jnp.int32), jnp.full((NPAD - N,), G, jnp.int32)]
    ).astype(f32).reshape(NPAD, 1)
    pad_n = E_PAD - E
    pad_src = (jnp.arange(pad_n, dtype=jnp.int32) * 17) % N
    pad_dst = jnp.full((pad_n,), jnp.int32(1 << 28))
    src = jnp.concatenate([edge_index[0].astype(jnp.int32), pad_src]).reshape(E2D, CHUNK)
    dst = jnp.concatenate([edge_index[1].astype(jnp.int32), pad_dst]).reshape(E2D, CHUNK)

    x, px = _xbuild(deg_f, bat_f)
    params = [(conv0_w, eps0, bn0_g, bn0_b),
              (conv1_w, eps1, bn1_g, bn1_b),
              (conv2_w, eps2, bn2_g, bn2_b)]
    h = x
    pooled = [px]
    for (w, eps, g, bb) in params:
        e11 = jnp.reshape(eps.astype(f32), (1, 1))
        aggv = _sc_agg(h, src, dst)
        y, mom = _stats(h, aggv, e11, w)
        h, pk = _layer_out(y, bat_f, mom, g.reshape(1, D), bb.reshape(1, D))
        pooled.append(pk)
    return _head(pooled[0], pooled[1], pooled[2], pooled[3],
                 cls_w1, cls_b1.reshape(1, CLS_HIDDEN),
                 cls_w2, cls_b2.reshape(1, NUM_CLASSES))


# speedup vs baseline: 7.0086x; 1.1327x over previous
import functools

import jax
import jax.numpy as jnp
from jax import lax
from jax.experimental import pallas as pl
from jax.experimental.pallas import tpu as pltpu
from jax.experimental.pallas import tpu_sc as plsc

N = 50000
D = 64
G = 128
NUM_CLASSES = 10
CLS_HIDDEN = 128

NPAD = 50176
HALF = 25088
ACC_ROWS = 26624
E = 800000
CHUNK = 128
SUBCH = 8
NBLK = 49
EPT = NBLK * SUBCH * CHUNK
E_PAD = EPT * 16
E2D = E_PAD // CHUNK
TROWS = EPT // CHUNK

BLK = 1024
GRID = NPAD // BLK


def _leaky(x):
    return jnp.where(x >= 0, x, 0.01 * x)


def _sc_agg(h, src2, dst2):
    mesh = plsc.VectorSubcoreMesh(core_axis_name="c", subcore_axis_name="s")

    @functools.partial(
        pl.kernel,
        mesh=mesh,
        compiler_params=pltpu.CompilerParams(use_tc_tiling_on_sc=False),
        out_type=jax.ShapeDtypeStruct((NPAD, D), jnp.float32),
        scratch_types=[
            pltpu.VMEM((2, SUBCH, CHUNK), jnp.int32),
            pltpu.VMEM((2, SUBCH, CHUNK), jnp.int32),
            pltpu.VMEM((2, CHUNK, D), jnp.float32),
            pltpu.VMEM_SHARED((ACC_ROWS, D), jnp.float32),
            pltpu.SemaphoreType.DMA,
            pltpu.SemaphoreType.DMA,
            pltpu.SemaphoreType.DMA,
            pltpu.SemaphoreType.DMA,
            pltpu.SemaphoreType.DMA,
        ],
    )
    def agg_kernel(h_hbm, src_hbm, dst_hbm, out_hbm, eidx, ldst, rows4, acc,
                   esem, g0, g1, s0, s1):
        gsems = [g0, g1]
        ssems = [s0, s1]
        c = lax.axis_index("c")
        s = lax.axis_index("s")
        lo = c * HALF
        hi = lo + HALF
        lane = lax.iota(jnp.int32, 16)

        def gather_start(eb, i, b):
            pltpu.async_copy(h_hbm.at[eidx.at[eb, i]], rows4.at[b], gsems[b])

        def gather_wait(b):
            pltpu.make_async_copy(h_hbm.at[eidx.at[0, 0]], rows4.at[b],
                                  gsems[b]).wait()

        def scatter_start(eb, i, b):
            pltpu.async_copy(rows4.at[b], acc.at[ldst.at[eb, i]], ssems[b],
                             add=True)

        def scatter_wait(b):
            pltpu.make_async_copy(rows4.at[b], acc.at[ldst.at[0, 0]],
                                  ssems[b]).wait()

        def edge_start(eb, row0):
            pltpu.async_copy(src_hbm.at[pl.ds(row0, SUBCH)], eidx.at[eb], esem)
            pltpu.async_copy(dst_hbm.at[pl.ds(row0, SUBCH)], ldst.at[eb], esem)

        def edge_wait():
            pltpu.make_async_copy(src_hbm.at[pl.ds(0, SUBCH)], eidx.at[0],
                                  esem).wait()
            pltpu.make_async_copy(dst_hbm.at[pl.ds(0, SUBCH)], ldst.at[0],
                                  esem).wait()

        def zrow(r, carry):
            for k in range(D // 16):
                rows4[0, r, pl.ds(16 * k, 16)] = jnp.zeros((16,), jnp.float32)
            return carry

        lax.fori_loop(0, CHUNK, zrow, 0)
        for j in range(ACC_ROWS // 16 // CHUNK):
            pltpu.sync_copy(rows4.at[0],
                            acc.at[pl.ds(s * (ACC_ROWS // 16) + j * CHUNK, CHUNK)])
        plsc.subcore_barrier()

        edge_start(0, s * TROWS)

        def outer(blk, carry):
            eb = lax.rem(blk, 2)
            edge_wait()
            for i in range(SUBCH):
                for k in range(CHUNK // 16):
                    dv = ldst[eb, i, pl.ds(16 * k, 16)]
                    owned = (dv >= lo) & (dv < hi)
                    tr = HALF + s * 96 + ((i * CHUNK + 16 * k + lane) % 96)
                    ldst[eb, i, pl.ds(16 * k, 16)] = jnp.where(owned, dv - lo, tr)
            for i in range(SUBCH):
                b = i % 2
                pb = 1 - b
                if i >= 2:
                    scatter_wait(b)
                else:
                    @pl.when(blk > 0)
                    def _():
                        scatter_wait(b)
                gather_start(eb, i, b)
                if i >= 1:
                    gather_wait(pb)
                    scatter_start(eb, i - 1, pb)
                else:
                    @pl.when(blk > 0)
                    def _():
                        gather_wait(pb)
                        scatter_start(1 - eb, SUBCH - 1, pb)
                if i == 2:
                    @pl.when(blk + 1 < NBLK)
                    def _():
                        edge_start(1 - eb, s * TROWS + (blk + 1) * SUBCH)
            return carry

        lax.fori_loop(0, NBLK, outer, 0)

        gather_wait(1)
        scatter_start(0, SUBCH - 1, 1)
        scatter_wait(0)
        scatter_wait(1)
        plsc.subcore_barrier()

        pltpu.sync_copy(
            acc.at[pl.ds(s * (HALF // 16), HALF // 16)],
            out_hbm.at[pl.ds(c * HALF + s * (HALF // 16), HALF // 16)],
        )

    return agg_kernel(h, src2, dst2)


def _xbuild(deg_f, bat_f):
    def kern(deg_ref, bat_ref, x_ref, px_ref):
        step = pl.program_id(0)
        d = deg_ref[...]
        cols = lax.broadcasted_iota(jnp.int32, (BLK, D), 1).astype(jnp.float32)
        x = (d == cols).astype(jnp.float32)
        x_ref[...] = x
        gcols = lax.broadcasted_iota(jnp.int32, (BLK, G), 1).astype(jnp.float32)
        p = (bat_ref[...] == gcols).astype(jnp.float32)

        @pl.when(step == 0)
        def _():
            px_ref[...] = jnp.zeros_like(px_ref)

        px_ref[...] += lax.dot_general(p, x, (((0,), (0,)), ((), ())),
                                       preferred_element_type=jnp.float32)

    return pl.pallas_call(
        kern,
        grid=(GRID,),
        in_specs=[pl.BlockSpec((BLK, 1), lambda i: (i, 0)),
                  pl.BlockSpec((BLK, 1), lambda i: (i, 0))],
        out_specs=[pl.BlockSpec((BLK, D), lambda i: (i, 0)),
                   pl.BlockSpec((G, D), lambda i: (0, 0))],
        out_shape=[jax.ShapeDtypeStruct((NPAD, D), jnp.float32),
                   jax.ShapeDtypeStruct((G, D), jnp.float32)],
    )(deg_f, bat_f)


def _stats(h, agg, eps11, w):
    def kern(h_ref, a_ref, e_ref, w_ref, y_ref, mom_ref):
        step = pl.program_id(0)
        u = (1.0 + e_ref[0, 0]) * h_ref[...] + a_ref[...]
        y = jnp.dot(u, w_ref[...], preferred_element_type=jnp.float32,
                    precision=lax.Precision.HIGHEST)
        y_ref[...] = y
        rows = lax.broadcasted_iota(jnp.int32, (BLK, 1), 0) + step * BLK
        ym = jnp.where(rows < N, y, 0.0)

        @pl.when(step == 0)
        def _():
            mom_ref[...] = jnp.zeros_like(mom_ref)

        riota = lax.broadcasted_iota(jnp.int32, (8, D), 0)
        row0 = (riota == 0).astype(jnp.float32)
        row1 = (riota == 1).astype(jnp.float32)
        mom_ref[...] += (row0 * jnp.sum(ym, axis=0, keepdims=True)
                         + row1 * jnp.sum(ym * ym, axis=0, keepdims=True))

    return pl.pallas_call(
        kern,
        grid=(GRID,),
        in_specs=[pl.BlockSpec((BLK, D), lambda i: (i, 0)),
                  pl.BlockSpec((BLK, D), lambda i: (i, 0)),
                  pl.BlockSpec((1, 1), lambda i: (0, 0)),
                  pl.BlockSpec((D, D), lambda i: (0, 0))],
        out_specs=[pl.BlockSpec((BLK, D), lambda i: (i, 0)),
                   pl.BlockSpec((8, D), lambda i: (0, 0))],
        out_shape=[jax.ShapeDtypeStruct((NPAD, D), jnp.float32),
                   jax.ShapeDtypeStruct((8, D), jnp.float32)],
    )(h, agg, eps11, w)


def _layer_out(y, bat_f, mom, g, bb):
    def kern(y_ref, bat_ref, mom_ref, g_ref, bb_ref, hn_ref, p_ref):
        step = pl.program_id(0)
        m = mom_ref[...]
        mean = m[0:1, :] / N
        var = m[1:2, :] / N - mean * mean
        scale = g_ref[...] * lax.rsqrt(var + 1e-5)
        z = _leaky((y_ref[...] - mean) * scale + bb_ref[...])
        hn_ref[...] = z
        gcols = lax.broadcasted_iota(jnp.int32, (BLK, G), 1).astype(jnp.float32)
        p = (bat_ref[...] == gcols).astype(jnp.float32)

        @pl.when(step == 0)
        def _():
            p_ref[...] = jnp.zeros_like(p_ref)

        p_ref[...] += lax.dot_general(p, z, (((0,), (0,)), ((), ())),
                                      preferred_element_type=jnp.float32,
                                      precision=lax.Precision.HIGHEST)

    return pl.pallas_call(
        kern,
        grid=(GRID,),
        in_specs=[pl.BlockSpec((BLK, D), lambda i: (i, 0)),
                  pl.BlockSpec((BLK, 1), lambda i: (i, 0)),
                  pl.BlockSpec((8, D), lambda i: (0, 0)),
                  pl.BlockSpec((1, D), lambda i: (0, 0)),
                  pl.BlockSpec((1, D), lambda i: (0, 0))],
        out_specs=[pl.BlockSpec((BLK, D), lambda i: (i, 0)),
                   pl.BlockSpec((G, D), lambda i: (0, 0))],
        out_shape=[jax.ShapeDtypeStruct((NPAD, D), jnp.float32),
                   jax.ShapeDtypeStruct((G, D), jnp.float32)],
    )(y, bat_f, mom, g, bb)


def _head(px, p1, p2, p3, w1, b1, w2, b2):
    def kern(px_ref, p1_ref, p2_ref, p3_ref, w1_ref, b1_ref, w2_ref, b2_ref, o_ref):
        o = jnp.dot(_leaky(px_ref[...]), w1_ref[0:D, :],
                    preferred_element_type=jnp.float32, precision=lax.Precision.HIGHEST)
        o += jnp.dot(_leaky(p1_ref[...]), w1_ref[D:2 * D, :],
                     preferred_element_type=jnp.float32, precision=lax.Precision.HIGHEST)
        o += jnp.dot(_leaky(p2_ref[...]), w1_ref[2 * D:3 * D, :],
                     preferred_element_type=jnp.float32, precision=lax.Precision.HIGHEST)
        o += jnp.dot(_leaky(p3_ref[...]), w1_ref[3 * D:4 * D, :],
                     preferred_element_type=jnp.float32, precision=lax.Precision.HIGHEST)
        o = _leaky(o + b1_ref[...])
        o_ref[...] = jnp.dot(o, w2_ref[...], preferred_element_type=jnp.float32, precision=lax.Precision.HIGHEST) \
            + b2_ref[...]

    return pl.pallas_call(
        kern,
        out_shape=jax.ShapeDtypeStruct((G, NUM_CLASSES), jnp.float32),
    )(px, p1, p2, p3, w1, b1, w2, b2)


def kernel(node_deg, edge_index, batch,
           conv0_w, conv0_b, eps0, bn0_g, bn0_b,
           conv1_w, conv1_b, eps1, bn1_g, bn1_b,
           conv2_w, conv2_b, eps2, bn2_g, bn2_b,
           cls_w1, cls_b1, cls_w2, cls_b2):
    f32 = jnp.float32
    deg = node_deg.astype(jnp.int32)
    deg_f = jnp.concatenate(
        [deg, jnp.full((NPAD - N,), -1, jnp.int32)]).astype(f32).reshape(NPAD, 1)
    bat_f = jnp.concatenate(
        [batch.astype(jnp.int32), jnp.full((NPAD - N,), G, jnp.int32)]
    ).astype(f32).reshape(NPAD, 1)
    pad_n = E_PAD - E
    pad_src = (jnp.arange(pad_n, dtype=jnp.int32) * 17) % N
    pad_dst = jnp.full((pad_n,), jnp.int32(1 << 28))
    src = jnp.concatenate([edge_index[0].astype(jnp.int32), pad_src]).reshape(E2D, CHUNK)
    dst = jnp.concatenate([edge_index[1].astype(jnp.int32), pad_dst]).reshape(E2D, CHUNK)

    x, px = _xbuild(deg_f, bat_f)
    params = [(conv0_w, eps0, bn0_g, bn0_b),
              (conv1_w, eps1, bn1_g, bn1_b),
              (conv2_w, eps2, bn2_g, bn2_b)]
    h = x
    pooled = [px]
    for (w, eps, g, bb) in params:
        e11 = jnp.reshape(eps.astype(f32), (1, 1))
        aggv = _sc_agg(h, src, dst)
        y, mom = _stats(h, aggv, e11, w)
        h, pk = _layer_out(y, bat_f, mom, g.reshape(1, D), bb.reshape(1, D))
        pooled.append(pk)
    return _head(pooled[0], pooled[1], pooled[2], pooled[3],
                 cls_w1, cls_b1.reshape(1, CLS_HIDDEN),
                 cls_w2, cls_b2.reshape(1, NUM_CLASSES))
